# baseline (speedup 1.0000x reference)
import functools

import jax
import jax.numpy as jnp
from jax import lax
from jax.experimental import pallas as pl
from jax.experimental.pallas import tpu as pltpu
from jax.experimental.pallas import tpu_sc as plsc

B = 16384
D = 8
H = 256
NC = 2
NS = 16
NW = NC * NS
BPW = B // NW
BBM = 4096

_DN = lax.GatherDimensionNumbers(offset_dims=(), collapsed_slice_dims=(0,),
                                 start_index_map=(0,))


def _perm(vec, idxvec):
    return lax.gather(vec, idxvec[:, None], _DN, (1,),
                      mode=lax.GatherScatterMode.PROMISE_IN_BOUNDS)


def _sc_body(idx_hbm, gt_hbm, ct_hbm, out_hbm, idx_v, gt_v, ct_v, out_v):
    c = lax.axis_index("c")
    s = lax.axis_index("s")
    wid = s * NC + c
    base = wid * BPW
    pltpu.sync_copy(gt_hbm, gt_v)
    pltpu.sync_copy(ct_hbm, ct_v)
    pltpu.sync_copy(idx_hbm.at[pl.ds(base * 32, BPW * 32)], idx_v)

    lane = lax.iota(jnp.int32, 16)
    tail8 = lane % 8
    lo = lane < 8

    def rows(src, a, b):
        return _perm(src, jnp.where(lo, a, b)) * 8 + tail8

    def mean10(rows_list):
        acc = rows_list[0]
        for r in rows_list[1:]:
            acc = acc + r
        up = _perm(acc, tail8 + 8)
        return (acc + up) * 0.1

    def body(i, carry):
        r0 = idx_v[pl.ds(i * 32, 16)]
        r1 = idx_v[pl.ds(i * 32 + 16, 16)]
        g01 = plsc.load_gather(gt_v, [rows(r0, 0, 1)])
        g23 = plsc.load_gather(gt_v, [rows(r0, 2, 3)])
        g44 = plsc.load_gather(gt_v, [rows(r0, 4, 4)])
        c01 = plsc.load_gather(ct_v, [rows(r0, 5, 6)])
        c23 = plsc.load_gather(ct_v, [rows(r0, 7, 8)])
        c44 = plsc.load_gather(ct_v, [rows(r0, 9, 9)])
        pres = mean10([
            plsc.load_gather(gt_v, [rows(r0, 10, 11)]),
            plsc.load_gather(gt_v, [rows(r0, 12, 13)]),
            plsc.load_gather(gt_v, [rows(r0, 14, 15)]),
            plsc.load_gather(gt_v, [rows(r1, 0, 1)]),
            plsc.load_gather(gt_v, [rows(r1, 2, 3)]),
        ])
        absm = mean10([
            plsc.load_gather(gt_v, [rows(r1, 4, 5)]),
            plsc.load_gather(gt_v, [rows(r1, 6, 7)]),
            plsc.load_gather(gt_v, [rows(r1, 8, 9)]),
            plsc.load_gather(gt_v, [rows(r1, 10, 11)]),
            plsc.load_gather(gt_v, [rows(r1, 12, 13)]),
        ])
        pres_hi = _perm(pres, tail8)
        abs_hi = _perm(absm, tail8)
        out_v[pl.ds(i * 96, 16)] = g01
        out_v[pl.ds(i * 96 + 16, 16)] = g23
        out_v[pl.ds(i * 96 + 32, 16)] = jnp.where(lo, g44, pres_hi)
        out_v[pl.ds(i * 96 + 48, 16)] = c01
        out_v[pl.ds(i * 96 + 64, 16)] = c23
        out_v[pl.ds(i * 96 + 80, 16)] = jnp.where(lo, c44, abs_hi)
        return carry

    lax.fori_loop(0, BPW, body, 0)
    pltpu.sync_copy(out_v, out_hbm.at[pl.ds(base * 96, BPW * 96)])


def _sc_combined(idx_flat, gt_flat, ct_flat):
    mesh = plsc.VectorSubcoreMesh(core_axis_name="c", subcore_axis_name="s")
    kfn = functools.partial(
        pl.kernel,
        mesh=mesh,
        compiler_params=pltpu.CompilerParams(needs_layout_passes=False),
        out_type=jax.ShapeDtypeStruct((B * 96,), jnp.float32),
        scratch_types=[
            pltpu.VMEM((BPW * 32,), jnp.int32),
            pltpu.VMEM((26 * D,), jnp.float32),
            pltpu.VMEM((27 * D,), jnp.float32),
            pltpu.VMEM((BPW * 96,), jnp.float32),
        ],
    )(_sc_body)
    return kfn(idx_flat, gt_flat, ct_flat)


def _mlp_body(comb_ref, w1_ref, b1_ref, w2_ref, b2_ref, out_ref, w1p_ref):
    @pl.when(pl.program_id(0) == 0)
    def _prep():
        w1 = w1_ref[:]
        w1p_ref[:] = jnp.concatenate(
            [w1[0:40], w1[80:88], w1[40:80], w1[88:96]], axis=0
        ).astype(jnp.bfloat16)

    comb = comb_ref[:].astype(jnp.bfloat16)
    h = jax.lax.dot(comb, w1p_ref[:], preferred_element_type=jnp.float32)
    h = jnp.maximum(h + b1_ref[:], 0.0).astype(jnp.bfloat16)
    out = jax.lax.dot(h, w2_ref[:], preferred_element_type=jnp.float32)
    out_ref[:] = out + b2_ref[:]


def _mlp(comb, W1, b1, W2, b2):
    b1r = b1.reshape(1, H).astype(jnp.bfloat16)
    w2r = W2.astype(jnp.bfloat16)
    b2r = b2.reshape(1, 1)
    return pl.pallas_call(
        _mlp_body,
        grid=(B // BBM,),
        in_specs=[
            pl.BlockSpec((BBM, 96), lambda i: (i, 0)),
            pl.BlockSpec((96, H), lambda i: (0, 0)),
            pl.BlockSpec((1, H), lambda i: (0, 0)),
            pl.BlockSpec((H, 1), lambda i: (0, 0)),
            pl.BlockSpec((1, 1), lambda i: (0, 0)),
        ],
        out_specs=pl.BlockSpec((BBM, 1), lambda i: (i, 0)),
        out_shape=jax.ShapeDtypeStruct((B, 1), jnp.float32),
        scratch_shapes=[pltpu.VMEM((96, H), jnp.bfloat16)],
    )(comb, W1, b1r, w2r, b2r)


@jax.jit
def kernel(guess_indices, constraint_indices, presence_list, absent_list,
           guess_table, constraint_table, W1, b1, W2, b2):
    idx = jnp.concatenate(
        [guess_indices, constraint_indices, presence_list, absent_list,
         jnp.zeros((B, 2), guess_indices.dtype)], axis=1).astype(jnp.int32)
    comb_flat = _sc_combined(idx.reshape(-1), guess_table.reshape(-1),
                             constraint_table.reshape(-1))
    return _mlp(comb_flat.reshape(B, 96), W1, b1, W2, b2)
